# Initial kernel scaffold; baseline (speedup 1.0000x reference)
import functools

import jax
import jax.numpy as jnp
from jax import lax
from jax.experimental import pallas as pl
from jax.experimental.pallas import tpu as pltpu
from jax.experimental.pallas import tpu_sc as plsc

N = 50000
B = 32
NW = 32
R = 1563
NPAD = NW * R
CH = 512
NSTREAM = CH // 128


@jax.jit
def _sc_run(xT, meta, starts):
    mesh = plsc.VectorSubcoreMesh(core_axis_name="c", subcore_axis_name="s")

    @functools.partial(
        pl.kernel,
        out_type=jax.ShapeDtypeStruct((NPAD * B,), jnp.float32),
        mesh=mesh,
        scratch_types=[
            pltpu.VMEM((R * B,), jnp.float32),
            pltpu.VMEM((4 * CH,), jnp.int32),
            pltpu.VMEM((CH, B), jnp.float32),
            pltpu.VMEM((40,), jnp.int32),
            pltpu.SemaphoreType.DMA,
            pltpu.SemaphoreType.DMA,
        ],
    )
    def sc_kernel(xT_hbm, meta_hbm, starts_hbm, out_hbm,
                  buf, metav, rows, stv, sem, gsem):
        wid = lax.axis_index("c") * 16 + lax.axis_index("s")
        base_row = wid * R

        zeros = jnp.zeros((16,), jnp.float32)

        @pl.loop(0, R * B, step=16)
        def _(i):
            buf[pl.ds(i, 16)] = zeros

        pltpu.sync_copy(starts_hbm, stv)
        e0 = stv[wid]
        e1 = stv[wid + 1]
        c0 = e0 // CH
        c1 = (e1 + CH - 1) // CH

        def chunk_body(c, carry):
            cb = c * CH
            pltpu.async_copy(meta_hbm.at[c], metav, sem).wait()
            for j in range(NSTREAM):
                pltpu.async_copy(
                    xT_hbm.at[metav.at[pl.ds(j * 128, 128)]],
                    rows.at[pl.ds(j * 128, 128)], gsem)
            for j in range(NSTREAM):
                pltpu.make_async_copy(
                    xT_hbm.at[metav.at[pl.ds(j * 128, 128)]],
                    rows.at[pl.ds(j * 128, 128)], gsem).wait()

            lo = jnp.maximum(e0 - cb, 0)
            hi = jnp.minimum(e1 - cb, CH)

            def edge_body(i, carry2):
                d = metav[CH + i]
                off = (d - base_row) * B
                ks = lax.bitcast_convert_type(metav[2 * CH + i], jnp.float32)
                bs = lax.bitcast_convert_type(metav[3 * CH + i], jnp.float32)
                kvec = jnp.full((16,), ks, jnp.float32)
                bvec = jnp.full((16,), bs, jnp.float32)
                r0 = rows[i, pl.ds(0, 16)]
                r1 = rows[i, pl.ds(16, 16)]
                plsc.addupdate(buf.at[pl.ds(off, 16)], r0 * kvec + bvec)
                plsc.addupdate(buf.at[pl.ds(off + 16, 16)], r1 * kvec + bvec)
                return carry2

            lax.fori_loop(lo, hi, edge_body, 0)
            return carry

        lax.fori_loop(c0, c1, chunk_body, 0)

        pltpu.sync_copy(buf, out_hbm.at[pl.ds(base_row * B, R * B)])

    return sc_kernel(xT, meta, starts)


def kernel(x, src_idx, dst_idx, kernel, bias):
    xT = x.T
    meta = jnp.concatenate(
        [src_idx.reshape(-1, CH),
         dst_idx.reshape(-1, CH),
         kernel.view(jnp.int32).reshape(-1, CH),
         bias.view(jnp.int32).reshape(-1, CH)],
        axis=1)
    bounds = jnp.arange(NW + 1, dtype=jnp.int32) * R
    starts = jnp.searchsorted(dst_idx, bounds, side="left").astype(jnp.int32)
    starts = jnp.concatenate([starts, jnp.zeros((7,), jnp.int32)])
    outf = _sc_run(xT, meta, starts)
    return outf.reshape(NPAD, B)[:N].T


# speedup vs baseline: 7.6974x; 7.6974x over previous
import functools

import jax
import jax.numpy as jnp
from jax import lax
from jax.experimental import pallas as pl
from jax.experimental.pallas import tpu as pltpu
from jax.experimental.pallas import tpu_sc as plsc

N = 50000
B = 32
NW = 32
R = 1563
NPAD = NW * R
CH = 256
NSTREAM = CH // 128


@jax.jit
def _sc_run(xT, gidx, smeta, starts):
    mesh = plsc.VectorSubcoreMesh(core_axis_name="c", subcore_axis_name="s")

    @functools.partial(
        pl.kernel,
        out_type=jax.ShapeDtypeStruct((NPAD * B,), jnp.float32),
        mesh=mesh,
        scratch_types=[
            pltpu.VMEM((R * B,), jnp.float32),
            pltpu.VMEM((CH,), jnp.int32),
            pltpu.VMEM((CH, B), jnp.float32),
            pltpu.VMEM((3 * CH + 16,), jnp.int32),
            pltpu.VMEM((48,), jnp.int32),
            pltpu.SemaphoreType.DMA,
            pltpu.SemaphoreType.DMA,
        ],
        compiler_params=pltpu.CompilerParams(use_tc_tiling_on_sc=False),
    )
    def sc_kernel(xT_hbm, gidx_hbm, smeta_hbm, starts_hbm, out_hbm,
                  buf, sidx, rows, sm, stv, sem, gsem):
        wid = lax.axis_index("c") * 16 + lax.axis_index("s")
        base_row = wid * R

        zeros = jnp.zeros((16,), jnp.float32)

        @pl.loop(0, R * B, step=16)
        def _(i):
            buf[pl.ds(i, 16)] = zeros

        pltpu.sync_copy(starts_hbm, stv.at[pl.ds(0, 40)])
        st16 = stv[pl.ds(wid, 16)]
        e0 = st16[0]
        e1 = st16[1]
        c0 = e0 // CH
        c1 = (e1 + CH - 1) // CH

        def chunk_body(c, carry):
            cb = c * CH
            pltpu.async_copy(gidx_hbm.at[c], sidx, sem)
            pltpu.async_copy(smeta_hbm.at[c], sm.at[pl.ds(0, 3 * CH)], sem)
            pltpu.make_async_copy(gidx_hbm.at[c], sidx, sem).wait()
            pltpu.make_async_copy(
                smeta_hbm.at[c], sm.at[pl.ds(0, 3 * CH)], sem).wait()
            for j in range(NSTREAM):
                pltpu.async_copy(
                    xT_hbm.at[sidx.at[pl.ds(j * 128, 128)]],
                    rows.at[pl.ds(j * 128, 128)], gsem)
            for j in range(NSTREAM):
                pltpu.make_async_copy(
                    xT_hbm.at[sidx.at[pl.ds(j * 128, 128)]],
                    rows.at[pl.ds(j * 128, 128)], gsem).wait()

            lo = jnp.maximum(e0 - cb, 0)
            hi = jnp.minimum(e1 - cb, CH)

            def edge_body(i, carry2):
                d = sm[pl.ds(i, 16)][0]
                off = (d - base_row) * B
                ks = lax.bitcast_convert_type(sm[pl.ds(CH + i, 16)][0],
                                              jnp.float32)
                bs = lax.bitcast_convert_type(sm[pl.ds(2 * CH + i, 16)][0],
                                              jnp.float32)
                kvec = jnp.full((16,), ks, jnp.float32)
                bvec = jnp.full((16,), bs, jnp.float32)
                r0 = rows[i, pl.ds(0, 16)]
                r1 = rows[i, pl.ds(16, 16)]
                plsc.addupdate(buf.at[pl.ds(off, 16)], r0 * kvec + bvec)
                plsc.addupdate(buf.at[pl.ds(off + 16, 16)], r1 * kvec + bvec)
                return carry2

            lax.fori_loop(lo, hi, edge_body, 0)
            return carry

        lax.fori_loop(c0, c1, chunk_body, 0)

        pltpu.sync_copy(buf, out_hbm.at[pl.ds(base_row * B, R * B)])

    return sc_kernel(xT, gidx, smeta, starts)


def kernel(x, src_idx, dst_idx, kernel, bias):
    xT = x.T
    gidx = src_idx.reshape(-1, CH)
    smeta = jnp.concatenate(
        [dst_idx.reshape(-1, CH),
         kernel.view(jnp.int32).reshape(-1, CH),
         bias.view(jnp.int32).reshape(-1, CH)],
        axis=1)
    bounds = jnp.arange(NW + 1, dtype=jnp.int32) * R
    starts = jnp.searchsorted(dst_idx, bounds, side="left").astype(jnp.int32)
    starts = jnp.concatenate([starts, jnp.zeros((7,), jnp.int32)])
    outf = _sc_run(xT, gidx, smeta, starts)
    return outf.reshape(NPAD, B)[:N].T


# speedup vs baseline: 9.1881x; 1.1937x over previous
import functools

import jax
import jax.numpy as jnp
from jax import lax
from jax.experimental import pallas as pl
from jax.experimental.pallas import tpu as pltpu
from jax.experimental.pallas import tpu_sc as plsc

N = 50000
B = 32
NW = 32
R = 1563
NPAD = NW * R
CH = 256
NSTREAM = CH // 128


@jax.jit
def _sc_run(xT, gidx, smeta, starts):
    mesh = plsc.VectorSubcoreMesh(core_axis_name="c", subcore_axis_name="s")

    @functools.partial(
        pl.kernel,
        out_type=jax.ShapeDtypeStruct((NPAD * B,), jnp.float32),
        mesh=mesh,
        scratch_types=[
            pltpu.VMEM((R * B,), jnp.float32),
            pltpu.VMEM((CH,), jnp.int32),
            pltpu.VMEM((CH,), jnp.int32),
            pltpu.VMEM((CH, B), jnp.float32),
            pltpu.VMEM((CH, B), jnp.float32),
            pltpu.VMEM((3 * CH + 16,), jnp.int32),
            pltpu.VMEM((3 * CH + 16,), jnp.int32),
            pltpu.VMEM((48,), jnp.int32),
            pltpu.SemaphoreType.DMA,
            pltpu.SemaphoreType.DMA,
            pltpu.SemaphoreType.DMA,
            pltpu.SemaphoreType.DMA,
        ],
        compiler_params=pltpu.CompilerParams(use_tc_tiling_on_sc=False),
    )
    def sc_kernel(xT_hbm, gidx_hbm, smeta_hbm, starts_hbm, out_hbm,
                  buf, sidx0, sidx1, rows0, rows1, sm0, sm1, stv,
                  msem0, msem1, gsem0, gsem1):
        wid = lax.axis_index("c") * 16 + lax.axis_index("s")
        base_row = wid * R

        zeros = jnp.zeros((16,), jnp.float32)

        @pl.loop(0, R * B, step=16)
        def _(i):
            buf[pl.ds(i, 16)] = zeros

        pltpu.sync_copy(starts_hbm, stv.at[pl.ds(0, 40)])
        st16 = stv[pl.ds(wid, 16)]
        e0 = st16[0]
        e1 = st16[1]
        c0 = e0 // CH
        c1 = (e1 + CH - 1) // CH

        def meta_issue(c, sidxb, smb, msem):
            @pl.when(c < c1)
            def _():
                pltpu.async_copy(gidx_hbm.at[c], sidxb, msem)
                pltpu.async_copy(smeta_hbm.at[c], smb.at[pl.ds(0, 3 * CH)],
                                 msem)

        def meta_wait(c, sidxb, smb, msem):
            @pl.when(c < c1)
            def _():
                pltpu.make_async_copy(gidx_hbm.at[c], sidxb, msem).wait()
                pltpu.make_async_copy(smeta_hbm.at[c],
                                      smb.at[pl.ds(0, 3 * CH)], msem).wait()

        def gather_issue(c, sidxb, rowsb, gsem):
            @pl.when(c < c1)
            def _():
                for j in range(NSTREAM):
                    pltpu.async_copy(
                        xT_hbm.at[sidxb.at[pl.ds(j * 128, 128)]],
                        rowsb.at[pl.ds(j * 128, 128)], gsem)

        def gather_wait(c, sidxb, rowsb, gsem):
            @pl.when(c < c1)
            def _():
                for j in range(NSTREAM):
                    pltpu.make_async_copy(
                        xT_hbm.at[sidxb.at[pl.ds(j * 128, 128)]],
                        rowsb.at[pl.ds(j * 128, 128)], gsem).wait()

        def compute(c, smb, rowsb):
            @pl.when(c < c1)
            def _():
                cb = c * CH
                lo = jnp.maximum(e0 - cb, 0)
                hi = jnp.minimum(e1 - cb, CH)

                def edge_body(i, carry2):
                    d = smb[pl.ds(i, 16)][0]
                    off = (d - base_row) * B
                    ks = lax.bitcast_convert_type(
                        smb[pl.ds(CH + i, 16)][0], jnp.float32)
                    bs = lax.bitcast_convert_type(
                        smb[pl.ds(2 * CH + i, 16)][0], jnp.float32)
                    kvec = jnp.full((16,), ks, jnp.float32)
                    bvec = jnp.full((16,), bs, jnp.float32)
                    r0 = rowsb[i, pl.ds(0, 16)]
                    r1 = rowsb[i, pl.ds(16, 16)]
                    plsc.addupdate(buf.at[pl.ds(off, 16)], r0 * kvec + bvec)
                    plsc.addupdate(buf.at[pl.ds(off + 16, 16)],
                                   r1 * kvec + bvec)
                    return carry2

                lax.fori_loop(lo, hi, edge_body, 0)

        meta_issue(c0, sidx0, sm0, msem0)
        meta_wait(c0, sidx0, sm0, msem0)
        gather_issue(c0, sidx0, rows0, gsem0)
        meta_issue(c0 + 1, sidx1, sm1, msem1)

        def pair_body(p, carry):
            a = c0 + 2 * p
            b = a + 1
            meta_wait(b, sidx1, sm1, msem1)
            gather_issue(b, sidx1, rows1, gsem1)
            gather_wait(a, sidx0, rows0, gsem0)
            compute(a, sm0, rows0)
            meta_issue(a + 2, sidx0, sm0, msem0)
            gather_wait(b, sidx1, rows1, gsem1)
            compute(b, sm1, rows1)
            meta_wait(a + 2, sidx0, sm0, msem0)
            gather_issue(a + 2, sidx0, rows0, gsem0)
            meta_issue(b + 2, sidx1, sm1, msem1)
            return carry

        npairs = (c1 - c0 + 1) // 2
        lax.fori_loop(0, npairs, pair_body, 0)

        pltpu.sync_copy(buf, out_hbm.at[pl.ds(base_row * B, R * B)])

    return sc_kernel(xT, gidx, smeta, starts)


def kernel(x, src_idx, dst_idx, kernel, bias):
    xT = x.T
    gidx = src_idx.reshape(-1, CH)
    smeta = jnp.concatenate(
        [dst_idx.reshape(-1, CH),
         kernel.view(jnp.int32).reshape(-1, CH),
         bias.view(jnp.int32).reshape(-1, CH)],
        axis=1)
    bounds = jnp.arange(NW + 1, dtype=jnp.int32) * R
    starts = jnp.searchsorted(dst_idx, bounds, side="left").astype(jnp.int32)
    starts = jnp.concatenate([starts, jnp.zeros((7,), jnp.int32)])
    outf = _sc_run(xT, gidx, smeta, starts)
    return outf.reshape(NPAD, B)[:N].T


# speedup vs baseline: 16.1621x; 1.7590x over previous
import functools

import jax
import jax.numpy as jnp
from jax import lax
from jax.experimental import pallas as pl
from jax.experimental.pallas import tpu as pltpu
from jax.experimental.pallas import tpu_sc as plsc

N = 50000
B = 32
NW = 32
R = 1563
NPAD = NW * R
CH = 512
NSTREAM = CH // 128


def _t_body(x_ref, o_ref):
    o_ref[...] = x_ref[...].T


def _tc_transpose_in(x):
    blk = 2048
    return pl.pallas_call(
        _t_body,
        out_shape=jax.ShapeDtypeStruct((N, B), jnp.float32),
        grid=((N + 2047) // 2048,),
        in_specs=[pl.BlockSpec((B, blk), lambda i: (0, i))],
        out_specs=pl.BlockSpec((blk, B), lambda i: (i, 0)),
    )(x)


def _tc_transpose_out(y):
    blk = 2048
    return pl.pallas_call(
        _t_body,
        out_shape=jax.ShapeDtypeStruct((B, N), jnp.float32),
        grid=((N + 2047) // 2048,),
        in_specs=[pl.BlockSpec((blk, B), lambda i: (i, 0))],
        out_specs=pl.BlockSpec((B, blk), lambda i: (0, i)),
    )(y)


@jax.jit
def _sc_run(xT, gidx, smeta, starts):
    mesh = plsc.VectorSubcoreMesh(core_axis_name="c", subcore_axis_name="s")

    @functools.partial(
        pl.kernel,
        out_type=jax.ShapeDtypeStruct((NPAD * B,), jnp.float32),
        mesh=mesh,
        scratch_types=[
            pltpu.VMEM((R * B + 32,), jnp.float32),
            pltpu.VMEM((CH,), jnp.int32),
            pltpu.VMEM((CH,), jnp.int32),
            pltpu.VMEM((CH, B), jnp.float32),
            pltpu.VMEM((CH, B), jnp.float32),
            pltpu.VMEM((4 * CH + 16,), jnp.int32),
            pltpu.VMEM((4 * CH + 16,), jnp.int32),
            pltpu.VMEM((48,), jnp.int32),
            pltpu.SemaphoreType.DMA,
            pltpu.SemaphoreType.DMA,
            pltpu.SemaphoreType.DMA,
            pltpu.SemaphoreType.DMA,
        ],
        compiler_params=pltpu.CompilerParams(use_tc_tiling_on_sc=False,
                                             needs_layout_passes=False),
    )
    def sc_kernel(xT_hbm, gidx_hbm, smeta_hbm, starts_hbm, out_hbm,
                  buf, sidx0, sidx1, rows0, rows1, sm0, sm1, stv,
                  msem0, msem1, gsem0, gsem1):
        wid = lax.axis_index("c") * 16 + lax.axis_index("s")
        base_row = wid * R

        zeros = jnp.zeros((16,), jnp.float32)

        @pl.loop(0, R * B + 32, step=16)
        def _(i):
            buf[pl.ds(i, 16)] = zeros

        pltpu.sync_copy(starts_hbm, stv.at[pl.ds(0, 40)])
        st16 = stv[pl.ds(wid, 16)]
        e0 = st16[0]
        e1 = st16[1]
        c0 = e0 // CH
        c1 = (e1 + CH - 1) // CH

        def meta_issue(c, sidxb, smb, msem):
            @pl.when(c < c1)
            def _():
                pltpu.async_copy(gidx_hbm.at[c], sidxb, msem)
                pltpu.async_copy(smeta_hbm.at[c], smb.at[pl.ds(0, 4 * CH)],
                                 msem)

        def meta_wait(c, sidxb, smb, msem):
            @pl.when(c < c1)
            def _():
                pltpu.make_async_copy(gidx_hbm.at[c], sidxb, msem).wait()
                pltpu.make_async_copy(smeta_hbm.at[c],
                                      smb.at[pl.ds(0, 4 * CH)], msem).wait()

        def gather_issue(c, sidxb, rowsb, gsem):
            @pl.when(c < c1)
            def _():
                for j in range(NSTREAM):
                    pltpu.async_copy(
                        xT_hbm.at[sidxb.at[pl.ds(j * 128, 128)]],
                        rowsb.at[pl.ds(j * 128, 128)], gsem)

        def gather_wait(c, sidxb, rowsb, gsem):
            @pl.when(c < c1)
            def _():
                for j in range(NSTREAM):
                    pltpu.make_async_copy(
                        xT_hbm.at[sidxb.at[pl.ds(j * 128, 128)]],
                        rowsb.at[pl.ds(j * 128, 128)], gsem).wait()

        iota16 = lax.iota(jnp.int32, 16)
        a0vec = iota16 - base_row * B
        a1vec = a0vec + 16
        e0v = jnp.full((16,), e0, jnp.int32)
        e1v = jnp.full((16,), e1, jnp.int32)
        dump0 = jnp.full((16,), R * B, jnp.int32) + iota16
        zf16 = jnp.zeros((16,), jnp.float32)

        def compute(c, smb, rowsb):
            cb = c * CH
            is_full = (c < c1) & (cb >= e0) & (cb + CH <= e1)

            @pl.when(is_full)
            def _():
                @plsc.parallel_loop(0, CH, 1, unroll=8)
                def _(g):
                    m16 = smb[pl.ds(4 * g, 16)]
                    dv = jnp.full((16,), m16[0], jnp.int32)
                    kvec = jnp.full(
                        (16,),
                        lax.bitcast_convert_type(m16[1], jnp.float32),
                        jnp.float32)
                    bvec = jnp.full(
                        (16,),
                        lax.bitcast_convert_type(m16[2], jnp.float32),
                        jnp.float32)
                    r0 = rowsb[g, pl.ds(0, 16)]
                    r1 = rowsb[g, pl.ds(16, 16)]
                    plsc.addupdate_scatter(buf, [dv + a0vec],
                                           r0 * kvec + bvec)
                    plsc.addupdate_scatter(buf, [dv + a1vec],
                                           r1 * kvec + bvec)

            @pl.when((c < c1) & jnp.logical_not(is_full))
            def _():
                def edge_body(p, carry2):
                    m16 = smb[pl.ds(4 * p, 16)]
                    ev = jnp.full((16,), m16[3], jnp.int32)
                    valid = (ev >= e0v) & (ev < e1v)
                    dv = jnp.full((16,), m16[0], jnp.int32)
                    aj = jnp.where(valid, dv + a0vec, dump0)
                    kvec = jnp.where(
                        valid,
                        jnp.full((16,),
                                 lax.bitcast_convert_type(m16[1],
                                                          jnp.float32),
                                 jnp.float32), zf16)
                    bvec = jnp.where(
                        valid,
                        jnp.full((16,),
                                 lax.bitcast_convert_type(m16[2],
                                                          jnp.float32),
                                 jnp.float32), zf16)
                    r0 = rowsb[p, pl.ds(0, 16)]
                    r1 = rowsb[p, pl.ds(16, 16)]
                    plsc.addupdate_scatter(buf, [aj], r0 * kvec + bvec)
                    plsc.addupdate_scatter(buf, [aj + 16], r1 * kvec + bvec)
                    return carry2

                lax.fori_loop(0, CH, edge_body, 0)

        meta_issue(c0, sidx0, sm0, msem0)
        meta_wait(c0, sidx0, sm0, msem0)
        gather_issue(c0, sidx0, rows0, gsem0)
        meta_issue(c0 + 1, sidx1, sm1, msem1)

        def pair_body(p, carry):
            a = c0 + 2 * p
            b = a + 1
            meta_wait(b, sidx1, sm1, msem1)
            gather_issue(b, sidx1, rows1, gsem1)
            gather_wait(a, sidx0, rows0, gsem0)
            compute(a, sm0, rows0)
            meta_issue(a + 2, sidx0, sm0, msem0)
            gather_wait(b, sidx1, rows1, gsem1)
            compute(b, sm1, rows1)
            meta_wait(a + 2, sidx0, sm0, msem0)
            gather_issue(a + 2, sidx0, rows0, gsem0)
            meta_issue(b + 2, sidx1, sm1, msem1)
            return carry

        npairs = (c1 - c0 + 1) // 2
        lax.fori_loop(0, npairs, pair_body, 0)

        pltpu.sync_copy(buf.at[pl.ds(0, R * B)],
                        out_hbm.at[pl.ds(base_row * B, R * B)])

    return sc_kernel(xT, gidx, smeta, starts)


def kernel(x, src_idx, dst_idx, kernel, bias):
    xT = _tc_transpose_in(x)
    eids = jnp.arange(src_idx.shape[0], dtype=jnp.int32)
    quad = jnp.stack(
        [dst_idx * B,
         kernel.view(jnp.int32),
         bias.view(jnp.int32),
         eids], axis=1)
    smeta = quad.reshape(-1, CH // 64, 64, 4).swapaxes(1, 2).reshape(
        -1, 4 * CH)
    gidx = src_idx.reshape(-1, CH // 64, 64).swapaxes(1, 2).reshape(-1, CH)
    bounds = jnp.arange(NW + 1, dtype=jnp.int32) * R
    starts = jnp.searchsorted(dst_idx, bounds, side="left").astype(jnp.int32)
    starts = jnp.concatenate([starts, jnp.zeros((7,), jnp.int32)])
    outf = _sc_run(xT, gidx, smeta, starts)
    return _tc_transpose_out(outf.reshape(NPAD, B))


# speedup vs baseline: 17.3496x; 1.0735x over previous
import functools

import jax
import jax.numpy as jnp
from jax import lax
from jax.experimental import pallas as pl
from jax.experimental.pallas import tpu as pltpu
from jax.experimental.pallas import tpu_sc as plsc

N = 50000
B = 32
NW = 32
R = 1563
NPAD = NW * R
CH = 512
NSTREAM = CH // 128


@jax.jit
def _sc_run(xT, gidx, smeta, starts):
    mesh = plsc.VectorSubcoreMesh(core_axis_name="c", subcore_axis_name="s")

    @functools.partial(
        pl.kernel,
        out_type=jax.ShapeDtypeStruct((NPAD * B,), jnp.float32),
        mesh=mesh,
        scratch_types=[
            pltpu.VMEM((R * B + 32,), jnp.float32),
            pltpu.VMEM((CH,), jnp.int32),
            pltpu.VMEM((CH,), jnp.int32),
            pltpu.VMEM((CH, B), jnp.float32),
            pltpu.VMEM((CH, B), jnp.float32),
            pltpu.VMEM((4 * CH + 16,), jnp.int32),
            pltpu.VMEM((4 * CH + 16,), jnp.int32),
            pltpu.VMEM((48,), jnp.int32),
            pltpu.SemaphoreType.DMA,
            pltpu.SemaphoreType.DMA,
            pltpu.SemaphoreType.DMA,
            pltpu.SemaphoreType.DMA,
        ],
        compiler_params=pltpu.CompilerParams(use_tc_tiling_on_sc=False,
                                             needs_layout_passes=False),
    )
    def sc_kernel(xT_hbm, gidx_hbm, smeta_hbm, starts_hbm, out_hbm,
                  buf, sidx0, sidx1, rows0, rows1, sm0, sm1, stv,
                  msem0, msem1, gsem0, gsem1):
        wid = lax.axis_index("c") * 16 + lax.axis_index("s")
        base_row = wid * R

        zeros = jnp.zeros((16,), jnp.float32)

        @pl.loop(0, R * B + 32, step=16)
        def _(i):
            buf[pl.ds(i, 16)] = zeros

        pltpu.sync_copy(starts_hbm, stv.at[pl.ds(0, 40)])
        st16 = stv[pl.ds(wid, 16)]
        e0 = st16[0]
        e1 = st16[1]
        c0 = e0 // CH
        c1 = (e1 + CH - 1) // CH

        def meta_issue(c, sidxb, smb, msem):
            @pl.when(c < c1)
            def _():
                pltpu.async_copy(gidx_hbm.at[c], sidxb, msem)
                pltpu.async_copy(smeta_hbm.at[c], smb.at[pl.ds(0, 4 * CH)],
                                 msem)

        def meta_wait(c, sidxb, smb, msem):
            @pl.when(c < c1)
            def _():
                pltpu.make_async_copy(gidx_hbm.at[c], sidxb, msem).wait()
                pltpu.make_async_copy(smeta_hbm.at[c],
                                      smb.at[pl.ds(0, 4 * CH)], msem).wait()

        def gather_issue(c, sidxb, rowsb, gsem):
            @pl.when(c < c1)
            def _():
                for j in range(NSTREAM):
                    pltpu.async_copy(
                        xT_hbm.at[sidxb.at[pl.ds(j * 128, 128)]],
                        rowsb.at[pl.ds(j * 128, 128)], gsem)

        def gather_wait(c, sidxb, rowsb, gsem):
            @pl.when(c < c1)
            def _():
                for j in range(NSTREAM):
                    pltpu.make_async_copy(
                        xT_hbm.at[sidxb.at[pl.ds(j * 128, 128)]],
                        rowsb.at[pl.ds(j * 128, 128)], gsem).wait()

        iota16 = lax.iota(jnp.int32, 16)
        a0vec = iota16 - base_row * B
        a1vec = a0vec + 16
        e0v = jnp.full((16,), e0, jnp.int32)
        e1v = jnp.full((16,), e1, jnp.int32)
        dump0 = jnp.full((16,), R * B, jnp.int32) + iota16
        zf16 = jnp.zeros((16,), jnp.float32)

        def compute(c, smb, rowsb):
            cb = c * CH
            is_full = (c < c1) & (cb >= e0) & (cb + CH <= e1)

            @pl.when(is_full)
            def _():
                @plsc.parallel_loop(0, CH, 1, unroll=8)
                def _(g):
                    m16 = smb[pl.ds(4 * g, 16)]
                    dv = jnp.full((16,), m16[0], jnp.int32)
                    kvec = jnp.full(
                        (16,),
                        lax.bitcast_convert_type(m16[1], jnp.float32),
                        jnp.float32)
                    bvec = jnp.full(
                        (16,),
                        lax.bitcast_convert_type(m16[2], jnp.float32),
                        jnp.float32)
                    r0 = rowsb[g, pl.ds(0, 16)]
                    r1 = rowsb[g, pl.ds(16, 16)]
                    plsc.addupdate_scatter(buf, [dv + a0vec],
                                           r0 * kvec + bvec)
                    plsc.addupdate_scatter(buf, [dv + a1vec],
                                           r1 * kvec + bvec)

            @pl.when((c < c1) & jnp.logical_not(is_full))
            def _():
                def edge_body(p, carry2):
                    m16 = smb[pl.ds(4 * p, 16)]
                    ev = jnp.full((16,), m16[3], jnp.int32)
                    valid = (ev >= e0v) & (ev < e1v)
                    dv = jnp.full((16,), m16[0], jnp.int32)
                    aj = jnp.where(valid, dv + a0vec, dump0)
                    kvec = jnp.where(
                        valid,
                        jnp.full((16,),
                                 lax.bitcast_convert_type(m16[1],
                                                          jnp.float32),
                                 jnp.float32), zf16)
                    bvec = jnp.where(
                        valid,
                        jnp.full((16,),
                                 lax.bitcast_convert_type(m16[2],
                                                          jnp.float32),
                                 jnp.float32), zf16)
                    r0 = rowsb[p, pl.ds(0, 16)]
                    r1 = rowsb[p, pl.ds(16, 16)]
                    plsc.addupdate_scatter(buf, [aj], r0 * kvec + bvec)
                    plsc.addupdate_scatter(buf, [aj + 16], r1 * kvec + bvec)
                    return carry2

                lax.fori_loop(0, CH, edge_body, 0)

        meta_issue(c0, sidx0, sm0, msem0)
        meta_wait(c0, sidx0, sm0, msem0)
        gather_issue(c0, sidx0, rows0, gsem0)
        meta_issue(c0 + 1, sidx1, sm1, msem1)

        def pair_body(p, carry):
            a = c0 + 2 * p
            b = a + 1
            meta_wait(b, sidx1, sm1, msem1)
            gather_issue(b, sidx1, rows1, gsem1)
            gather_wait(a, sidx0, rows0, gsem0)
            compute(a, sm0, rows0)
            meta_issue(a + 2, sidx0, sm0, msem0)
            gather_wait(b, sidx1, rows1, gsem1)
            compute(b, sm1, rows1)
            meta_wait(a + 2, sidx0, sm0, msem0)
            gather_issue(a + 2, sidx0, rows0, gsem0)
            meta_issue(b + 2, sidx1, sm1, msem1)
            return carry

        npairs = (c1 - c0 + 1) // 2
        lax.fori_loop(0, npairs, pair_body, 0)

        pltpu.sync_copy(buf.at[pl.ds(0, R * B)],
                        out_hbm.at[pl.ds(base_row * B, R * B)])

    return sc_kernel(xT, gidx, smeta, starts)


def kernel(x, src_idx, dst_idx, kernel, bias):
    xT = x.T
    eids = jnp.arange(src_idx.shape[0], dtype=jnp.int32)
    quad = jnp.stack(
        [dst_idx * B,
         kernel.view(jnp.int32),
         bias.view(jnp.int32),
         eids], axis=1)
    smeta = quad.reshape(-1, CH // 64, 64, 4).swapaxes(1, 2).reshape(
        -1, 4 * CH)
    gidx = src_idx.reshape(-1, CH // 64, 64).swapaxes(1, 2).reshape(-1, CH)
    bounds = jnp.arange(NW + 1, dtype=jnp.int32) * R
    starts = jnp.searchsorted(dst_idx, bounds, side="left",
                          method="scan_unrolled").astype(jnp.int32)
    starts = jnp.concatenate([starts, jnp.zeros((7,), jnp.int32)])
    outf = _sc_run(xT, gidx, smeta, starts)
    return outf.reshape(NPAD, B)[:N].T


# speedup vs baseline: 17.9832x; 1.0365x over previous
import functools

import jax
import jax.numpy as jnp
from jax import lax
from jax.experimental import pallas as pl
from jax.experimental.pallas import tpu as pltpu
from jax.experimental.pallas import tpu_sc as plsc

N = 50000
B = 32
NW = 32
R = 1563
NPAD = NW * R
CH = 512
NSTREAM = CH // 128


@jax.jit
def _sc_run(xT, gidx, smeta, starts):
    mesh = plsc.VectorSubcoreMesh(core_axis_name="c", subcore_axis_name="s")

    @functools.partial(
        pl.kernel,
        out_type=jax.ShapeDtypeStruct((NPAD * B,), jnp.float32),
        mesh=mesh,
        scratch_types=[
            pltpu.VMEM((R * B + 32,), jnp.float32),
            pltpu.VMEM((CH,), jnp.int32),
            pltpu.VMEM((CH,), jnp.int32),
            pltpu.VMEM((CH, B), jnp.float32),
            pltpu.VMEM((CH, B), jnp.float32),
            pltpu.VMEM((4 * CH + 16,), jnp.int32),
            pltpu.VMEM((4 * CH + 16,), jnp.int32),
            pltpu.VMEM((48,), jnp.int32),
            pltpu.SemaphoreType.DMA,
            pltpu.SemaphoreType.DMA,
            pltpu.SemaphoreType.DMA,
            pltpu.SemaphoreType.DMA,
        ],
        compiler_params=pltpu.CompilerParams(use_tc_tiling_on_sc=False,
                                             needs_layout_passes=False),
    )
    def sc_kernel(xT_hbm, gidx_hbm, smeta_hbm, starts_hbm, out_hbm,
                  buf, sidx0, sidx1, rows0, rows1, sm0, sm1, stv,
                  msem0, msem1, gsem0, gsem1):
        wid = lax.axis_index("c") * 16 + lax.axis_index("s")
        base_row = wid * R

        zeros = jnp.zeros((16,), jnp.float32)

        @pl.loop(0, R * B + 32, step=16)
        def _(i):
            buf[pl.ds(i, 16)] = zeros

        pltpu.sync_copy(starts_hbm, stv.at[pl.ds(0, 40)])
        st16 = stv[pl.ds(wid, 16)]
        e0 = st16[0]
        e1 = st16[1]
        c0 = e0 // CH
        c1 = (e1 + CH - 1) // CH

        def meta_issue(c, sidxb, smb, msem):
            @pl.when(c < c1)
            def _():
                pltpu.async_copy(gidx_hbm.at[c], sidxb, msem)
                pltpu.async_copy(smeta_hbm.at[c], smb.at[pl.ds(0, 4 * CH)],
                                 msem)

        def meta_wait(c, sidxb, smb, msem):
            @pl.when(c < c1)
            def _():
                pltpu.make_async_copy(gidx_hbm.at[c], sidxb, msem).wait()
                pltpu.make_async_copy(smeta_hbm.at[c],
                                      smb.at[pl.ds(0, 4 * CH)], msem).wait()

        def gather_issue(c, sidxb, rowsb, gsem):
            @pl.when(c < c1)
            def _():
                for j in range(NSTREAM):
                    pltpu.async_copy(
                        xT_hbm.at[sidxb.at[pl.ds(j * 128, 128)]],
                        rowsb.at[pl.ds(j * 128, 128)], gsem)

        def gather_wait(c, sidxb, rowsb, gsem):
            @pl.when(c < c1)
            def _():
                for j in range(NSTREAM):
                    pltpu.make_async_copy(
                        xT_hbm.at[sidxb.at[pl.ds(j * 128, 128)]],
                        rowsb.at[pl.ds(j * 128, 128)], gsem).wait()

        iota16 = lax.iota(jnp.int32, 16)
        a0vec = iota16 - base_row * B
        a1vec = a0vec + 16
        e0v = jnp.full((16,), e0, jnp.int32)
        e1v = jnp.full((16,), e1, jnp.int32)
        dump0 = jnp.full((16,), R * B, jnp.int32) + iota16
        zf16 = jnp.zeros((16,), jnp.float32)

        def compute(c, smb, rowsb):
            cb = c * CH
            is_full = (c < c1) & (cb >= e0) & (cb + CH <= e1)

            @pl.when(is_full)
            def _():
                @plsc.parallel_loop(0, CH, 4, unroll=2)
                def _(q):
                    m16 = smb[pl.ds(4 * q, 16)]
                    for j in range(4):
                        dv = jnp.full((16,), m16[4 * j], jnp.int32)
                        kvec = jnp.full(
                            (16,),
                            lax.bitcast_convert_type(m16[4 * j + 1],
                                                     jnp.float32),
                            jnp.float32)
                        bvec = jnp.full(
                            (16,),
                            lax.bitcast_convert_type(m16[4 * j + 2],
                                                     jnp.float32),
                            jnp.float32)
                        r0 = rowsb[q + j, pl.ds(0, 16)]
                        r1 = rowsb[q + j, pl.ds(16, 16)]
                        plsc.addupdate_scatter(buf, [dv + a0vec],
                                               r0 * kvec + bvec)
                        plsc.addupdate_scatter(buf, [dv + a1vec],
                                               r1 * kvec + bvec)

            @pl.when((c < c1) & jnp.logical_not(is_full))
            def _():
                def edge_body(p, carry2):
                    m16 = smb[pl.ds(4 * p, 16)]
                    ev = jnp.full((16,), m16[3], jnp.int32)
                    valid = (ev >= e0v) & (ev < e1v)
                    dv = jnp.full((16,), m16[0], jnp.int32)
                    aj = jnp.where(valid, dv + a0vec, dump0)
                    kvec = jnp.where(
                        valid,
                        jnp.full((16,),
                                 lax.bitcast_convert_type(m16[1],
                                                          jnp.float32),
                                 jnp.float32), zf16)
                    bvec = jnp.where(
                        valid,
                        jnp.full((16,),
                                 lax.bitcast_convert_type(m16[2],
                                                          jnp.float32),
                                 jnp.float32), zf16)
                    r0 = rowsb[p, pl.ds(0, 16)]
                    r1 = rowsb[p, pl.ds(16, 16)]
                    plsc.addupdate_scatter(buf, [aj], r0 * kvec + bvec)
                    plsc.addupdate_scatter(buf, [aj + 16], r1 * kvec + bvec)
                    return carry2

                lax.fori_loop(0, CH, edge_body, 0)

        meta_issue(c0, sidx0, sm0, msem0)
        meta_wait(c0, sidx0, sm0, msem0)
        gather_issue(c0, sidx0, rows0, gsem0)
        meta_issue(c0 + 1, sidx1, sm1, msem1)

        def pair_body(p, carry):
            a = c0 + 2 * p
            b = a + 1
            meta_wait(b, sidx1, sm1, msem1)
            gather_issue(b, sidx1, rows1, gsem1)
            gather_wait(a, sidx0, rows0, gsem0)
            compute(a, sm0, rows0)
            meta_issue(a + 2, sidx0, sm0, msem0)
            gather_wait(b, sidx1, rows1, gsem1)
            compute(b, sm1, rows1)
            meta_wait(a + 2, sidx0, sm0, msem0)
            gather_issue(a + 2, sidx0, rows0, gsem0)
            meta_issue(b + 2, sidx1, sm1, msem1)
            return carry

        npairs = (c1 - c0 + 1) // 2
        lax.fori_loop(0, npairs, pair_body, 0)

        pltpu.sync_copy(buf.at[pl.ds(0, R * B)],
                        out_hbm.at[pl.ds(base_row * B, R * B)])

    return sc_kernel(xT, gidx, smeta, starts)


def kernel(x, src_idx, dst_idx, kernel, bias):
    xT = x.T
    eids = jnp.arange(src_idx.shape[0], dtype=jnp.int32)
    quad = jnp.stack(
        [dst_idx * B,
         kernel.view(jnp.int32),
         bias.view(jnp.int32),
         eids], axis=1)
    smeta = quad.reshape(-1, CH // 64, 64, 4).swapaxes(1, 2).reshape(
        -1, 4 * CH)
    gidx = src_idx.reshape(-1, CH // 64, 64).swapaxes(1, 2).reshape(-1, CH)
    bounds = jnp.arange(NW + 1, dtype=jnp.int32) * R
    starts = jnp.searchsorted(dst_idx, bounds, side="left",
                          method="scan_unrolled").astype(jnp.int32)
    starts = jnp.concatenate([starts, jnp.zeros((7,), jnp.int32)])
    outf = _sc_run(xT, gidx, smeta, starts)
    return outf.reshape(NPAD, B)[:N].T
